# Initial kernel scaffold; baseline (speedup 1.0000x reference)
import jax
import jax.numpy as jnp
from jax.experimental import pallas as pl


def kernel(x, edge_attr, W_x, b_x, W_e, b_e, mlp1_w, mlp1_b, bn_g, bn_b, mlp2_w, mlp2_b, gin_eps, ln_g, ln_b, edge_index):
    raise NotImplementedError("write your pallas kernel here")


# speedup vs baseline: 1.6483x; 1.6483x over previous
import functools

import jax
import jax.numpy as jnp
from jax import lax
from jax.experimental import pallas as pl
from jax.experimental.pallas import tpu as pltpu
from jax.experimental.pallas import tpu_sc as plsc

N = 10000
E = 320000
DIN = 128
DH = 128
DE = 16
L = 4
EPS = 1e-5

NC = 2
NS = 16
NW = NC * NS
EPW = E // NW
C = 80
NCHUNK = EPW // C
NPAD = 10240
RPT = NPAD // NS
ZR = 128


def _sc_message_pass(h, ea, srcs, dsts, perm):
    mesh = plsc.VectorSubcoreMesh(core_axis_name="c", subcore_axis_name="s")

    @functools.partial(
        pl.kernel,
        out_type=jax.ShapeDtypeStruct((NC, NPAD, DH), jnp.float32),
        mesh=mesh,
        scratch_types=[
            pltpu.VMEM((C,), jnp.int32),
            pltpu.VMEM((C,), jnp.int32),
            pltpu.VMEM((C,), jnp.int32),
            pltpu.VMEM((C, DH), jnp.float32),
            pltpu.VMEM((C, DH), jnp.float32),
            pltpu.VMEM((ZR, DH), jnp.float32),
            pltpu.VMEM((16,), jnp.int32),
            pltpu.VMEM_SHARED((NPAD, DH), jnp.float32),
            pltpu.SemaphoreType.DMA,
        ],
    )
    def k(h_hbm, ea_hbm, src_hbm, dst_hbm, perm_hbm, out_hbm,
          srcv, dstv, permv, gbuf, eabuf, zbuf, bndv, aggsh, sem):
        c = lax.axis_index("c")
        s = lax.axis_index("s")

        @pl.loop(0, ZR * (DH // 16))
        def _(i):
            zbuf[i // (DH // 16), pl.ds((i % (DH // 16)) * 16, 16)] = (
                jnp.zeros((16,), jnp.float32))

        row0 = pl.multiple_of(s * RPT, ZR)

        @pl.loop(0, RPT // ZR)
        def _(t):
            pltpu.sync_copy(
                zbuf, aggsh.at[pl.ds(pl.multiple_of(row0 + t * ZR, ZR), ZR)])

        plsc.subcore_barrier()

        wid = c * NS + s
        base_w = wid * EPW

        pltpu.sync_copy(
            dst_hbm.at[pl.ds(
                pl.multiple_of(jnp.maximum(base_w - 16, 0), 8), 16)], bndv)
        prev = jnp.where(wid == 0, jnp.int32(-1), bndv[...][15])

        def chunk_body(base, own_eq, bound):
            pltpu.sync_copy(src_hbm.at[pl.ds(base, C)], srcv)
            pltpu.sync_copy(dst_hbm.at[pl.ds(base, C)], dstv)
            pltpu.sync_copy(perm_hbm.at[pl.ds(base, C)], permv)
            cp_h = pltpu.async_copy(h_hbm.at[srcv], gbuf, sem)
            cp_e = pltpu.async_copy(ea_hbm.at[permv], eabuf, sem)
            cp_h.wait()
            cp_e.wait()

            @pl.loop(0, C // 16)
            def _(g):
                dvec = dstv[pl.ds(g * 16, 16)]
                kv = (dvec == bound) if own_eq else (dvec != bound)
                fv = jnp.where(kv, 1.0, 0.0).astype(jnp.float32)
                for ii in range(16):
                    i = g * 16 + ii
                    keep = fv[ii]
                    for kk in range(DH // 16):
                        sl = pl.ds(kk * 16, 16)
                        m = jnp.maximum(gbuf[i, sl] + eabuf[i, sl], 0.0)
                        gbuf[i, sl] = m * keep

            pltpu.sync_copy(gbuf, aggsh.at[dstv], add=True)

        @pl.loop(0, NCHUNK)
        def _(j):
            chunk_body(pl.multiple_of(base_w + j * C, 8), False, prev)

        pltpu.sync_copy(
            dst_hbm.at[pl.ds(pl.multiple_of(base_w + EPW - 16, 8), 16)],
            bndv)
        lastnode = bndv[...][15]

        @pl.when(wid < NW - 1)
        def _():
            @pl.loop(0, 2)
            def _(j):
                chunk_body(pl.multiple_of(base_w + EPW + j * C, 8), True,
                           lastnode)

        plsc.subcore_barrier()
        pltpu.sync_copy(aggsh.at[pl.ds(row0, RPT)],
                        out_hbm.at[c].at[pl.ds(row0, RPT)])

    return k(h, ea, srcs, dsts, perm)


def _embed_h_body(x_ref, w_ref, b_ref, o_ref):
    z = lax.dot_general(x_ref[...].astype(jnp.bfloat16),
                        w_ref[...].astype(jnp.bfloat16),
                        (((1,), (1,)), ((), ())),
                        preferred_element_type=jnp.float32)
    o_ref[...] = jnp.maximum(z + b_ref[...], 0.0)


def _embed_h(x, W_x, b_x):
    return pl.pallas_call(
        _embed_h_body,
        out_shape=jax.ShapeDtypeStruct((N, DH), jnp.float32),
    )(x, W_x, b_x.reshape(1, DH))


EB = 8000


def _embed_ea(edge_attr, W_e, b_e):
    return pl.pallas_call(
        _embed_h_body,
        grid=(E // EB,),
        in_specs=[
            pl.BlockSpec((EB, DE), lambda i: (i, 0)),
            pl.BlockSpec((DH, DE), lambda i: (0, 0)),
            pl.BlockSpec((1, DH), lambda i: (0, 0)),
        ],
        out_specs=pl.BlockSpec((EB, DH), lambda i: (i, 0)),
        out_shape=jax.ShapeDtypeStruct((E, DH), jnp.float32),
    )(edge_attr, W_e, b_e.reshape(1, DH))


def _mlp_body(last, h_ref, agg_ref, w1_ref, b1_ref, g_ref, bb_ref,
              w2_ref, b2_ref, eps_ref, lng_ref, lnb_ref, o_ref):
    h = h_ref[...]
    agg = agg_ref[0, :N, :] + agg_ref[1, :N, :]
    z = (1.0 + eps_ref[0, 0]) * h + agg
    z1 = lax.dot_general(z.astype(jnp.bfloat16),
                         w1_ref[...].astype(jnp.bfloat16),
                         (((1,), (1,)), ((), ())),
                         preferred_element_type=jnp.float32) + b1_ref[...]
    mu = jnp.mean(z1, axis=0, keepdims=True)
    var = jnp.mean((z1 - mu) ** 2, axis=0, keepdims=True)
    z1 = (z1 - mu) / jnp.sqrt(var + EPS) * g_ref[...] + bb_ref[...]
    z1 = jnp.maximum(z1, 0.0)
    z2 = lax.dot_general(z1.astype(jnp.bfloat16),
                         w2_ref[...].astype(jnp.bfloat16),
                         (((1,), (1,)), ((), ())),
                         preferred_element_type=jnp.float32) + b2_ref[...]
    mu2 = jnp.mean(z2, axis=1, keepdims=True)
    var2 = jnp.mean((z2 - mu2) ** 2, axis=1, keepdims=True)
    z2 = (z2 - mu2) / jnp.sqrt(var2 + EPS) * lng_ref[...] + lnb_ref[...]
    if not last:
        z2 = jnp.maximum(z2, 0.0)
    o_ref[...] = z2


def _mlp(last, h, agg, w1, b1, g, bb, w2, b2, eps, lng, lnb):
    return pl.pallas_call(
        functools.partial(_mlp_body, last),
        out_shape=jax.ShapeDtypeStruct((N, DH), jnp.float32),
    )(h, agg, w1, b1.reshape(1, 2 * DH), g.reshape(1, 2 * DH),
      bb.reshape(1, 2 * DH), w2, b2.reshape(1, DH), eps.reshape(1, 1),
      lng.reshape(1, DH), lnb.reshape(1, DH))


def kernel(x, edge_attr, W_x, b_x, W_e, b_e, mlp1_w, mlp1_b, bn_g, bn_b,
           mlp2_w, mlp2_b, gin_eps, ln_g, ln_b, edge_index):
    src = edge_index[0]
    dst = edge_index[1]
    perm = jnp.argsort(dst, stable=True).astype(jnp.int32)
    srcs = src[perm]
    dsts = dst[perm]
    h = _embed_h(x, W_x, b_x)
    ea = _embed_ea(edge_attr, W_e, b_e)
    for l in range(L):
        agg = _sc_message_pass(h, ea, srcs, dsts, perm)
        h = _mlp(l == L - 1, h, agg, mlp1_w[l], mlp1_b[l], bn_g[l], bn_b[l],
                 mlp2_w[l], mlp2_b[l], gin_eps[l], ln_g[l], ln_b[l])
    return h
